# baseline (speedup 1.0000x reference)
import functools

import jax
import jax.numpy as jnp
from jax import lax
from jax.experimental import pallas as pl
from jax.experimental.pallas import tpu as pltpu

_N = 20000
_K = 20
_NK = _N + _K
_NBLK = 20
_NPAD = _NBLK * 1024
_FG = 64
_NROI = 256
_BG = _NROI - _FG


def _ptl_kernel(box_ref, gtq_ref, out_ref,
                sfg_ref, sbg_ref, lab_ref, a1_ref, a2_ref, a3_ref, a4_ref):
    f32 = jnp.float32
    subi = lax.broadcasted_iota(jnp.int32, (8, 128), 0)
    lanei = lax.broadcasted_iota(jnp.int32, (8, 128), 1)
    io1024 = subi * 128 + lanei
    lane128 = lax.broadcasted_iota(jnp.int32, (1, 128), 1)
    lane256 = lax.broadcasted_iota(jnp.int32, (1, 256), 1)

    G = gtq_ref[0]
    def _gts(q, k):
        return jnp.sum(jnp.where((subi == q) & (lanei == k), G, 0.0))
    gx1 = [_gts(0, k) for k in range(_K)]
    gy1 = [_gts(1, k) for k in range(_K)]
    gx2 = [_gts(2, k) for k in range(_K)]
    gy2 = [_gts(3, k) for k in range(_K)]
    glab = [_gts(4, k) for k in range(_K)]

    def blk_body(blk, carry):
        bmf, bmb = carry
        bx1 = box_ref[0, 0, blk]
        by1 = box_ref[0, 1, blk]
        bx2 = box_ref[0, 2, blk]
        by2 = box_ref[0, 3, blk]
        area_b = (bx2 - bx1 + 1.0) * (by2 - by1 + 1.0)
        cur = jnp.full((8, 128), -1.0, f32)
        labv = jnp.zeros((8, 128), f32)
        a1 = jnp.zeros((8, 128), f32)
        a2 = jnp.zeros((8, 128), f32)
        a3 = jnp.zeros((8, 128), f32)
        a4 = jnp.zeros((8, 128), f32)
        for k in range(_K):
            iw = jnp.clip(jnp.minimum(bx2, gx2[k]) - jnp.maximum(bx1, gx1[k]) + 1.0, 0.0)
            ih = jnp.clip(jnp.minimum(by2, gy2[k]) - jnp.maximum(by1, gy1[k]) + 1.0, 0.0)
            inter = iw * ih
            area_g = (gx2[k] - gx1[k] + 1.0) * (gy2[k] - gy1[k] + 1.0)
            iou = inter / (area_b + area_g - inter)
            upd = iou > cur
            cur = jnp.where(upd, iou, cur)
            labv = jnp.where(upd, glab[k], labv)
            a1 = jnp.where(upd, gx1[k], a1)
            a2 = jnp.where(upd, gy1[k], a2)
            a3 = jnp.where(upd, gx2[k], a3)
            a4 = jnp.where(upd, gy2[k], a4)
        gidx = blk * 1024 + io1024
        valid = gidx < _NK
        fgs = jnp.where(valid, jnp.where(cur >= 0.5, cur, -1.0), -4.0)
        bgs = jnp.where(valid,
                        jnp.where((cur < 0.5) & (cur >= 0.1), cur, -1.0), -4.0)
        sfg_ref[blk] = fgs
        sbg_ref[blk] = bgs
        lab_ref[blk] = labv
        a1_ref[blk] = a1
        a2_ref[blk] = a2
        a3_ref[blk] = a3
        a4_ref[blk] = a4
        bmf = jnp.where(lane128 == blk, jnp.max(fgs), bmf)
        bmb = jnp.where(lane128 == blk, jnp.max(bgs), bmb)
        return bmf, bmb

    bm_init = jnp.full((1, 128), -5.0, f32)
    bmf, bmb = lax.fori_loop(0, _NBLK, blk_body, (bm_init, bm_init))

    def make_body(s_ref, slot_offset, is_fg):
        def body(i, carry):
            bm, px1, py1, px2, py2, plab, pa1, pa2, pa3, pa4 = carry
            m = jnp.max(bm)
            b = jnp.min(jnp.where(bm == m, lane128, 9999))
            chunk = s_ref[b]
            iin = jnp.min(jnp.where(chunk == m, io1024, 99999))
            onehot = io1024 == iin
            x1v = jnp.sum(jnp.where(onehot, box_ref[0, 0, b], 0.0))
            y1v = jnp.sum(jnp.where(onehot, box_ref[0, 1, b], 0.0))
            x2v = jnp.sum(jnp.where(onehot, box_ref[0, 2, b], 0.0))
            y2v = jnp.sum(jnp.where(onehot, box_ref[0, 3, b], 0.0))
            a1v = jnp.sum(jnp.where(onehot, a1_ref[b], 0.0))
            a2v = jnp.sum(jnp.where(onehot, a2_ref[b], 0.0))
            a3v = jnp.sum(jnp.where(onehot, a3_ref[b], 0.0))
            a4v = jnp.sum(jnp.where(onehot, a4_ref[b], 0.0))
            newchunk = jnp.where(onehot, -4.0, chunk)
            s_ref[b] = newchunk
            bm = jnp.where(lane128 == b, jnp.max(newchunk), bm)
            so = lane256 == (slot_offset + i)
            px1 = jnp.where(so, x1v, px1)
            py1 = jnp.where(so, y1v, py1)
            px2 = jnp.where(so, x2v, px2)
            py2 = jnp.where(so, y2v, py2)
            pa1 = jnp.where(so, a1v, pa1)
            pa2 = jnp.where(so, a2v, pa2)
            pa3 = jnp.where(so, a3v, pa3)
            pa4 = jnp.where(so, a4v, pa4)
            if is_fg:
                labv = jnp.sum(jnp.where(onehot, lab_ref[b], 0.0))
                labw = jnp.where(m > -0.5, labv, 0.0)
                plab = jnp.where(so, labw, plab)
            return bm, px1, py1, px2, py2, plab, pa1, pa2, pa3, pa4
        return body

    z = jnp.zeros((1, 256), f32)
    carry = (bmf, z, z, z, z, z, z, z, z, z)
    carry = lax.fori_loop(0, _FG, make_body(sfg_ref, 0, True), carry)
    carry = (bmb,) + carry[1:]
    carry = lax.fori_loop(0, _BG, make_body(sbg_ref, _FG, False), carry)
    _, px1, py1, px2, py2, plab, pa1, pa2, pa3, pa4 = carry

    ew = px2 - px1 + 1.0
    eh = py2 - py1 + 1.0
    ecx = px1 + 0.5 * ew
    ecy = py1 + 0.5 * eh
    gw = pa3 - pa1 + 1.0
    gh = pa4 - pa2 + 1.0
    gcx = pa1 + 0.5 * gw
    gcy = pa2 + 0.5 * gh
    tx = ((gcx - ecx) / ew) / 0.1
    ty = ((gcy - ecy) / eh) / 0.1
    tw = jnp.log(gw / ew) / 0.2
    th = jnp.log(gh / eh) / 0.2
    pos = plab > 0.0
    tx = jnp.where(pos, tx, 0.0)
    ty = jnp.where(pos, ty, 0.0)
    tw = jnp.where(pos, tw, 0.0)
    th = jnp.where(pos, th, 0.0)
    zz = jnp.zeros((7, 256), f32)
    out_ref[0] = jnp.concatenate(
        [px1, py1, px2, py2, plab, tx, ty, tw, th, zz], axis=0)


@jax.jit
def kernel(all_rois, gt_boxes, num_boxes):
    del num_boxes
    f32 = jnp.float32
    B, N, _ = all_rois.shape
    _, K, F, _ = gt_boxes.shape
    BF = B * F

    rois_xy = jnp.broadcast_to(all_rois[:, None], (B, F, N, 5))
    rois_xy = jnp.transpose(rois_xy, (0, 2, 1, 3)).reshape(BF, N, 5)
    gt = jnp.transpose(gt_boxes, (0, 2, 1, 3)).reshape(BF, K, 5)
    boxes = jnp.concatenate([rois_xy[:, :, 1:5], gt[:, :, 0:4]], axis=1)

    planes = jnp.transpose(boxes, (0, 2, 1))
    planes = jnp.pad(planes, ((0, 0), (0, 0), (0, _NPAD - _NK)))
    planes = planes.reshape(BF, 4, _NBLK, 8, 128)

    gtq = jnp.transpose(gt, (0, 2, 1))
    gtq = jnp.pad(gtq, ((0, 0), (0, 3), (0, 128 - K)))

    out = pl.pallas_call(
        _ptl_kernel,
        grid=(BF,),
        in_specs=[
            pl.BlockSpec((1, 4, _NBLK, 8, 128), lambda r: (r, 0, 0, 0, 0)),
            pl.BlockSpec((1, 8, 128), lambda r: (r, 0, 0)),
        ],
        out_specs=pl.BlockSpec((1, 16, 256), lambda r: (r, 0, 0)),
        out_shape=jax.ShapeDtypeStruct((BF, 16, 256), f32),
        scratch_shapes=[pltpu.VMEM((_NBLK, 8, 128), f32) for _ in range(7)],
        compiler_params=pltpu.CompilerParams(
            dimension_semantics=("parallel",)),
    )(planes, gtq)

    labels_batch = out[:, 4, :]
    batch_col = jnp.broadcast_to(
        jnp.arange(BF, dtype=f32)[:, None], (BF, _NROI))
    rois_batch = jnp.stack(
        [batch_col, out[:, 0, :], out[:, 1, :], out[:, 2, :], out[:, 3, :]],
        axis=-1)
    bbox_targets = jnp.transpose(out[:, 5:9, :], (0, 2, 1))
    pos4 = jnp.broadcast_to((labels_batch > 0.0)[:, :, None], (BF, _NROI, 4))
    bbox_inside = jnp.where(pos4, 1.0, 0.0).astype(f32)
    bbox_outside = (bbox_inside > 0).astype(f32)
    return rois_batch, labels_batch, bbox_targets, bbox_inside, bbox_outside


# speedup vs baseline: 1.0048x; 1.0048x over previous
import functools

import jax
import jax.numpy as jnp
from jax import lax
from jax.experimental import pallas as pl
from jax.experimental.pallas import tpu as pltpu

_N = 20000
_K = 20
_NK = _N + _K
_NBLK = 20
_NPAD = _NBLK * 1024
_FG = 64
_NROI = 256
_BG = _NROI - _FG


def _ptl_kernel(box_ref, gtq_ref, out_ref,
                sfg_ref, sbg_ref, lab_ref, a1_ref, a2_ref, a3_ref, a4_ref):
    f32 = jnp.float32
    subi = lax.broadcasted_iota(jnp.int32, (8, 128), 0)
    lanei = lax.broadcasted_iota(jnp.int32, (8, 128), 1)
    io1024 = subi * 128 + lanei
    lane128 = lax.broadcasted_iota(jnp.int32, (1, 128), 1)
    lane256 = lax.broadcasted_iota(jnp.int32, (1, 256), 1)

    G = gtq_ref[0]
    def _gts(q, k):
        return jnp.sum(jnp.where((subi == q) & (lanei == k), G, 0.0))
    gx1 = [_gts(0, k) for k in range(_K)]
    gy1 = [_gts(1, k) for k in range(_K)]
    gx2 = [_gts(2, k) for k in range(_K)]
    gy2 = [_gts(3, k) for k in range(_K)]
    glab = [_gts(4, k) for k in range(_K)]

    def blk_body(blk, carry):
        bmf, bmb = carry
        bx1 = box_ref[0, 0, blk]
        by1 = box_ref[0, 1, blk]
        bx2 = box_ref[0, 2, blk]
        by2 = box_ref[0, 3, blk]
        area_b = (bx2 - bx1 + 1.0) * (by2 - by1 + 1.0)
        cur = jnp.full((8, 128), -1.0, f32)
        labv = jnp.zeros((8, 128), f32)
        a1 = jnp.zeros((8, 128), f32)
        a2 = jnp.zeros((8, 128), f32)
        a3 = jnp.zeros((8, 128), f32)
        a4 = jnp.zeros((8, 128), f32)
        for k in range(_K):
            iw = jnp.clip(jnp.minimum(bx2, gx2[k]) - jnp.maximum(bx1, gx1[k]) + 1.0, 0.0)
            ih = jnp.clip(jnp.minimum(by2, gy2[k]) - jnp.maximum(by1, gy1[k]) + 1.0, 0.0)
            inter = iw * ih
            area_g = (gx2[k] - gx1[k] + 1.0) * (gy2[k] - gy1[k] + 1.0)
            iou = inter / (area_b + area_g - inter)
            upd = iou > cur
            cur = jnp.where(upd, iou, cur)
            labv = jnp.where(upd, glab[k], labv)
            a1 = jnp.where(upd, gx1[k], a1)
            a2 = jnp.where(upd, gy1[k], a2)
            a3 = jnp.where(upd, gx2[k], a3)
            a4 = jnp.where(upd, gy2[k], a4)
        gidx = blk * 1024 + io1024
        valid = gidx < _NK
        fgs = jnp.where(valid, jnp.where(cur >= 0.5, cur, -1.0), -4.0)
        bgs = jnp.where(valid,
                        jnp.where((cur < 0.5) & (cur >= 0.1), cur, -1.0), -4.0)
        sfg_ref[blk] = fgs
        sbg_ref[blk] = bgs
        lab_ref[blk] = labv
        a1_ref[blk] = a1
        a2_ref[blk] = a2
        a3_ref[blk] = a3
        a4_ref[blk] = a4
        bmf = jnp.where(lane128 == blk, jnp.max(fgs), bmf)
        bmb = jnp.where(lane128 == blk, jnp.max(bgs), bmb)
        return bmf, bmb

    bm_init = jnp.full((1, 128), -5.0, f32)
    bmf, bmb = lax.fori_loop(0, _NBLK, blk_body, (bm_init, bm_init))

    def make_body(s_ref, slot_offset, is_fg):
        def body(i, carry):
            bm, px1, py1, px2, py2, plab, pa1, pa2, pa3, pa4 = carry
            m = jnp.max(bm)
            b = jnp.min(jnp.where(bm == m, lane128, 9999))
            chunk = s_ref[b]
            iin = jnp.min(jnp.where(chunk == m, io1024, 99999))
            onehot = io1024 == iin
            x1v = jnp.sum(jnp.where(onehot, box_ref[0, 0, b], 0.0))
            y1v = jnp.sum(jnp.where(onehot, box_ref[0, 1, b], 0.0))
            x2v = jnp.sum(jnp.where(onehot, box_ref[0, 2, b], 0.0))
            y2v = jnp.sum(jnp.where(onehot, box_ref[0, 3, b], 0.0))
            newchunk = jnp.where(onehot, -4.0, chunk)
            s_ref[b] = newchunk
            bm = jnp.where(lane128 == b, jnp.max(newchunk), bm)
            so = lane256 == (slot_offset + i)
            px1 = jnp.where(so, x1v, px1)
            py1 = jnp.where(so, y1v, py1)
            px2 = jnp.where(so, x2v, px2)
            py2 = jnp.where(so, y2v, py2)
            if is_fg:
                a1v = jnp.sum(jnp.where(onehot, a1_ref[b], 0.0))
                a2v = jnp.sum(jnp.where(onehot, a2_ref[b], 0.0))
                a3v = jnp.sum(jnp.where(onehot, a3_ref[b], 0.0))
                a4v = jnp.sum(jnp.where(onehot, a4_ref[b], 0.0))
                pa1 = jnp.where(so, a1v, pa1)
                pa2 = jnp.where(so, a2v, pa2)
                pa3 = jnp.where(so, a3v, pa3)
                pa4 = jnp.where(so, a4v, pa4)
                labv = jnp.sum(jnp.where(onehot, lab_ref[b], 0.0))
                labw = jnp.where(m > -0.5, labv, 0.0)
                plab = jnp.where(so, labw, plab)
            return bm, px1, py1, px2, py2, plab, pa1, pa2, pa3, pa4
        return body

    z = jnp.zeros((1, 256), f32)
    carry = (bmf, z, z, z, z, z, z, z, z, z)
    carry = lax.fori_loop(0, _FG, make_body(sfg_ref, 0, True), carry)
    carry = (bmb,) + carry[1:]
    carry = lax.fori_loop(0, _BG, make_body(sbg_ref, _FG, False), carry)
    _, px1, py1, px2, py2, plab, pa1, pa2, pa3, pa4 = carry

    ew = px2 - px1 + 1.0
    eh = py2 - py1 + 1.0
    ecx = px1 + 0.5 * ew
    ecy = py1 + 0.5 * eh
    gw = pa3 - pa1 + 1.0
    gh = pa4 - pa2 + 1.0
    gcx = pa1 + 0.5 * gw
    gcy = pa2 + 0.5 * gh
    tx = ((gcx - ecx) / ew) / 0.1
    ty = ((gcy - ecy) / eh) / 0.1
    tw = jnp.log(gw / ew) / 0.2
    th = jnp.log(gh / eh) / 0.2
    pos = plab > 0.0
    tx = jnp.where(pos, tx, 0.0)
    ty = jnp.where(pos, ty, 0.0)
    tw = jnp.where(pos, tw, 0.0)
    th = jnp.where(pos, th, 0.0)
    zz = jnp.zeros((7, 256), f32)
    out_ref[0] = jnp.concatenate(
        [px1, py1, px2, py2, plab, tx, ty, tw, th, zz], axis=0)


@jax.jit
def kernel(all_rois, gt_boxes, num_boxes):
    del num_boxes
    f32 = jnp.float32
    B, N, _ = all_rois.shape
    _, K, F, _ = gt_boxes.shape
    BF = B * F

    rois_xy = jnp.broadcast_to(all_rois[:, None], (B, F, N, 5))
    rois_xy = jnp.transpose(rois_xy, (0, 2, 1, 3)).reshape(BF, N, 5)
    gt = jnp.transpose(gt_boxes, (0, 2, 1, 3)).reshape(BF, K, 5)
    boxes = jnp.concatenate([rois_xy[:, :, 1:5], gt[:, :, 0:4]], axis=1)

    planes = jnp.transpose(boxes, (0, 2, 1))
    planes = jnp.pad(planes, ((0, 0), (0, 0), (0, _NPAD - _NK)))
    planes = planes.reshape(BF, 4, _NBLK, 8, 128)

    gtq = jnp.transpose(gt, (0, 2, 1))
    gtq = jnp.pad(gtq, ((0, 0), (0, 3), (0, 128 - K)))

    out = pl.pallas_call(
        _ptl_kernel,
        grid=(BF,),
        in_specs=[
            pl.BlockSpec((1, 4, _NBLK, 8, 128), lambda r: (r, 0, 0, 0, 0)),
            pl.BlockSpec((1, 8, 128), lambda r: (r, 0, 0)),
        ],
        out_specs=pl.BlockSpec((1, 16, 256), lambda r: (r, 0, 0)),
        out_shape=jax.ShapeDtypeStruct((BF, 16, 256), f32),
        scratch_shapes=[pltpu.VMEM((_NBLK, 8, 128), f32) for _ in range(7)],
        compiler_params=pltpu.CompilerParams(
            dimension_semantics=("parallel",)),
    )(planes, gtq)

    labels_batch = out[:, 4, :]
    batch_col = jnp.broadcast_to(
        jnp.arange(BF, dtype=f32)[:, None], (BF, _NROI))
    rois_batch = jnp.stack(
        [batch_col, out[:, 0, :], out[:, 1, :], out[:, 2, :], out[:, 3, :]],
        axis=-1)
    bbox_targets = jnp.transpose(out[:, 5:9, :], (0, 2, 1))
    pos4 = jnp.broadcast_to((labels_batch > 0.0)[:, :, None], (BF, _NROI, 4))
    bbox_inside = jnp.where(pos4, 1.0, 0.0).astype(f32)
    bbox_outside = (bbox_inside > 0).astype(f32)
    return rois_batch, labels_batch, bbox_targets, bbox_inside, bbox_outside
